# baseline (speedup 1.0000x reference)
import functools

import jax
import jax.numpy as jnp
from jax import lax
from jax.experimental import pallas as pl
from jax.experimental.pallas import tpu as pltpu
from jax.experimental.pallas import tpu_sc as plsc

_B = 16384
_D = 32
_NC = 2
_NS = 16
_NW = _NC * _NS
_BPW = _B // _NW
_CH = 16

_mesh = plsc.VectorSubcoreMesh(core_axis_name="c", subcore_axis_name="s")


@functools.partial(
    pl.kernel,
    out_type=(
        jax.ShapeDtypeStruct((_B, _D), jnp.float32),
        jax.ShapeDtypeStruct((_B, _D), jnp.float32),
        jax.ShapeDtypeStruct((_B, _D), jnp.float32),
    ),
    mesh=_mesh,
    scratch_types=[
        pltpu.VMEM((_BPW,), jnp.int32),
        pltpu.VMEM((_BPW,), jnp.int32),
        pltpu.VMEM((_BPW,), jnp.int32),
        pltpu.SemaphoreType.DMA,
        pltpu.SemaphoreType.DMA,
        pltpu.SemaphoreType.DMA,
    ],
)
def _gather3(src_t, edge_t, dst_t, src_i, edge_i, dst_i,
             src_o, edge_o, dst_o,
             i0, i1, i2, s0, s1, s2):
    wid = lax.axis_index("s") * _NC + lax.axis_index("c")
    base = wid * _BPW

    pltpu.sync_copy(src_i.at[pl.ds(base, _BPW)], i0)
    pltpu.sync_copy(edge_i.at[pl.ds(base, _BPW)], i1)
    pltpu.sync_copy(dst_i.at[pl.ds(base, _BPW)], i2)

    def walk(table, idx_v, sem, out, wait):
        @pl.loop(0, _BPW // _CH)
        def chunk(c):
            cbase = c * _CH
            vec = idx_v[pl.ds(cbase, _CH)]
            for j in range(_CH):
                cp = pltpu.make_async_copy(
                    table.at[vec[j]], out.at[base + cbase + j], sem)
                if wait:
                    cp.wait()
                else:
                    cp.start()

    walk(src_t, i0, s0, src_o, False)
    walk(edge_t, i1, s1, edge_o, False)
    walk(dst_t, i2, s2, dst_o, False)
    walk(src_t, i0, s0, src_o, True)
    walk(edge_t, i1, s1, edge_o, True)
    walk(dst_t, i2, s2, dst_o, True)


def kernel(src_table, edge_table, dst_table, src_ids, edge_ids, dst_ids, offset):
    src_emb, edge_emb, dst_emb = _gather3(
        src_table, edge_table, dst_table,
        src_ids.astype(jnp.int32),
        edge_ids.astype(jnp.int32),
        dst_ids.astype(jnp.int32),
    )
    return (src_emb, edge_emb, dst_emb, offset)


# speedup vs baseline: 1.7888x; 1.7888x over previous
import functools

import jax
import jax.numpy as jnp
from jax import lax
from jax.experimental import pallas as pl
from jax.experimental.pallas import tpu as pltpu
from jax.experimental.pallas import tpu_sc as plsc

_B = 16384
_D = 32
_NC = 2
_NS = 16
_NW = _NC * _NS
_BPW = _B // _NW
_C = 16
_NCH = _BPW // _C

_mesh = plsc.VectorSubcoreMesh(core_axis_name="c", subcore_axis_name="s")


@functools.partial(
    pl.kernel,
    out_type=(
        jax.ShapeDtypeStruct((_B, _D), jnp.float32),
        jax.ShapeDtypeStruct((_B, _D), jnp.float32),
        jax.ShapeDtypeStruct((_B, _D), jnp.float32),
    ),
    mesh=_mesh,
    scratch_types=[
        pltpu.VMEM((_BPW,), jnp.int32),
        pltpu.VMEM((_BPW,), jnp.int32),
        pltpu.VMEM((2, _C, 8, _D), jnp.float32),
        pltpu.VMEM((_BPW, _D), jnp.float32),
        pltpu.SemaphoreType.DMA,
        pltpu.SemaphoreType.DMA,
    ],
)
def _gather3(src_t, edge_t, dst_t, src_i, edge_i, dst_i,
             src_o, edge_o, dst_o,
             idx_v, gidx_v, rows_b, out_v, gsem, osem):
    wid = lax.axis_index("s") * _NC + lax.axis_index("c")
    base = wid * _BPW

    def one_table(table, ids, out):
        pltpu.sync_copy(ids.at[pl.ds(base, _BPW)], idx_v)
        tv = table.reshape(table.shape[0] // 8, 8, _D)

        @pl.loop(0, _BPW // 16)
        def gidx(c):
            gidx_v[pl.ds(c * 16, 16)] = lax.shift_right_logical(
                idx_v[pl.ds(c * 16, 16)], 3)

        def gather(c, phase):
            vec_g = gidx_v[pl.ds(c * _C, _C)]
            return [
                pltpu.make_async_copy(
                    tv.at[vec_g[j]], rows_b.at[phase, j], gsem)
                for j in range(_C)
            ]

        for cp in gather(0, 0):
            cp.start()

        @pl.loop(0, _NCH)
        def chunk(c):
            p = lax.rem(c, 2)
            for cp in gather(c, p):
                cp.wait()

            @pl.when(c + 1 < _NCH)
            def _():
                for cp in gather(c + 1, 1 - p):
                    cp.start()

            cb = c * _C
            vec = idx_v[pl.ds(cb, _C)]
            for j in range(_C):
                slot = lax.rem(vec[j], 8)
                out_v[cb + j, pl.ds(0, 16)] = rows_b[p, j, slot, pl.ds(0, 16)]
                out_v[cb + j, pl.ds(16, 16)] = rows_b[p, j, slot, pl.ds(16, 16)]

        pltpu.async_copy(out_v, out.at[pl.ds(base, _BPW)], osem).wait()

    one_table(src_t, src_i, src_o)
    one_table(edge_t, edge_i, edge_o)
    one_table(dst_t, dst_i, dst_o)


def kernel(src_table, edge_table, dst_table, src_ids, edge_ids, dst_ids, offset):
    src_emb, edge_emb, dst_emb = _gather3(
        src_table, edge_table, dst_table,
        src_ids.astype(jnp.int32),
        edge_ids.astype(jnp.int32),
        dst_ids.astype(jnp.int32),
    )
    return (src_emb, edge_emb, dst_emb, offset)


# speedup vs baseline: 1.9134x; 1.0696x over previous
import functools

import jax
import jax.numpy as jnp
from jax import lax
from jax.experimental import pallas as pl
from jax.experimental.pallas import tpu as pltpu
from jax.experimental.pallas import tpu_sc as plsc

_B = 16384
_D = 32
_NC = 2
_NS = 16
_NW = _NC * _NS
_BPW = _B // _NW
_C = 16
_NCH = _BPW // _C

_mesh = plsc.VectorSubcoreMesh(core_axis_name="c", subcore_axis_name="s")


@functools.partial(
    pl.kernel,
    out_type=(
        jax.ShapeDtypeStruct((_B, _D), jnp.float32),
        jax.ShapeDtypeStruct((_B, _D), jnp.float32),
        jax.ShapeDtypeStruct((_B, _D), jnp.float32),
    ),
    mesh=_mesh,
    scratch_types=[
        pltpu.VMEM((_BPW,), jnp.int32),
        pltpu.VMEM((_BPW,), jnp.int32),
        pltpu.VMEM((2, _C, 8, _D), jnp.float32),
        pltpu.VMEM((_BPW, _D), jnp.float32),
        pltpu.SemaphoreType.DMA,
        pltpu.SemaphoreType.DMA,
    ],
)
def _gather3(src_t, edge_t, dst_t, src_i, edge_i, dst_i,
             src_o, edge_o, dst_o,
             idx_v, gidx_v, rows_b, out_v, gsem, osem):
    wid = lax.axis_index("s") * _NC + lax.axis_index("c")
    base = wid * _BPW

    def one_table(table, ids, out):
        pltpu.sync_copy(ids.at[pl.ds(base, _BPW)], idx_v)
        tv = table.reshape(table.shape[0] // 8, 8, _D)

        @pl.loop(0, _BPW // 16)
        def gidx(c):
            gidx_v[pl.ds(c * 16, 16)] = lax.shift_right_logical(
                idx_v[pl.ds(c * 16, 16)], 3)

        def gather(c, phase):
            vec_g = gidx_v[pl.ds(c * _C, _C)]
            return [
                pltpu.make_async_copy(
                    tv.at[vec_g[j]], rows_b.at[phase, j], gsem)
                for j in range(_C)
            ]

        for cp in gather(0, 0):
            cp.start()

        @pl.loop(0, _NCH)
        def chunk(c):
            p = lax.rem(c, 2)

            @pl.when(c + 1 < _NCH)
            def _():
                for cp in gather(c + 1, 1 - p):
                    cp.start()

            for cp in gather(c, p):
                cp.wait()

            cb = c * _C
            vec = idx_v[pl.ds(cb, _C)]
            for j in range(_C):
                slot = lax.rem(vec[j], 8)
                out_v[cb + j, pl.ds(0, 16)] = rows_b[p, j, slot, pl.ds(0, 16)]
                out_v[cb + j, pl.ds(16, 16)] = rows_b[p, j, slot, pl.ds(16, 16)]

        pltpu.async_copy(out_v, out.at[pl.ds(base, _BPW)], osem).wait()

    one_table(src_t, src_i, src_o)
    one_table(edge_t, edge_i, edge_o)
    one_table(dst_t, dst_i, dst_o)


def kernel(src_table, edge_table, dst_table, src_ids, edge_ids, dst_ids, offset):
    src_emb, edge_emb, dst_emb = _gather3(
        src_table, edge_table, dst_table,
        src_ids.astype(jnp.int32),
        edge_ids.astype(jnp.int32),
        dst_ids.astype(jnp.int32),
    )
    return (src_emb, edge_emb, dst_emb, offset)
